# baseline (speedup 1.0000x reference)
import jax
import jax.numpy as jnp
from jax import lax
from jax.experimental import pallas as pl
from jax.experimental.pallas import tpu as pltpu
from jax.experimental.pallas import tpu_sc as plsc

_info = plsc.get_sparse_core_info()
NC, NS = _info.num_cores, _info.num_subcores
NW = NC * NS

B = 4096 * 50
D = 128
ROWS_PER_W = B // NW
CH = 128
NCH = ROWS_PER_W // CH
NBUF = 5


def _body(tok_hbm, w_hbm, out_hbm, idx_v, rows_v, *sems):
  gsems, wsems = sems[:NBUF], sems[NBUF:]
  wid = lax.axis_index("s") * NC + lax.axis_index("c")
  base = wid * ROWS_PER_W

  pltpu.sync_copy(tok_hbm.at[wid], idx_v)

  def start_gather(c, b):
    pltpu.async_copy(w_hbm.at[idx_v.at[c]], rows_v.at[b], gsems[b])

  def wb_desc(c, b):
    return pltpu.make_async_copy(
        rows_v.at[b], out_hbm.at[pl.ds(base + c * CH, CH)], wsems[b])

  for b in range(NBUF):
    start_gather(b, b)

  @pl.loop(0, NCH, step=NBUF)
  def _(g0):
    for b in range(NBUF):
      g = g0 + b
      pltpu.make_async_copy(w_hbm.at[idx_v.at[g]], rows_v.at[b], gsems[b]).wait()
      wb_desc(g, b).start()
      pb = (b - 1) % NBUF

      @pl.when(g > 0)
      def _():
        wb_desc(g - 1, pb).wait()
      nxt = g - 1 + NBUF

      @pl.when((g > 0) & (nxt < NCH))
      def _():
        start_gather(nxt, pb)

  wb_desc(NCH - 1, (NCH - 1) % NBUF).wait()


@jax.jit
def kernel(token_ids, weight):
  tok = token_ids.reshape(NW, NCH, CH).astype(jnp.int32)
  mesh = plsc.VectorSubcoreMesh(core_axis_name="c", subcore_axis_name="s")
  call = pl.kernel(
      _body,
      out_type=jax.ShapeDtypeStruct((B, D), jnp.float32),
      mesh=mesh,
      scratch_types=[
          pltpu.VMEM((NCH, CH), jnp.int32),
          pltpu.VMEM((NBUF, CH, D), jnp.float32),
      ] + [pltpu.SemaphoreType.DMA] * (2 * NBUF),
      compiler_params=pltpu.CompilerParams(use_tc_tiling_on_sc=True),
  )
  out = call(tok, weight)
  return out.reshape(token_ids.shape[0], token_ids.shape[1], D)


# speedup vs baseline: 1.7848x; 1.7848x over previous
import jax
import jax.numpy as jnp
from jax import lax
from jax.experimental import pallas as pl
from jax.experimental.pallas import tpu as pltpu
from jax.experimental.pallas import tpu_sc as plsc

_info = plsc.get_sparse_core_info()
NC, NS = _info.num_cores, _info.num_subcores
NW = NC * NS

NSAMP = 4096
SEQ = 50
D = 128
SAMP_PER_W = NSAMP // NW
S = 2
CH_IDX = S * SEQ
NCH = SAMP_PER_W // S
NBUF = 4


def _body(tok_hbm, w_hbm, out_hbm, idx_v, rows_v, *sems):
  gsems, wsems = sems[:NBUF], sems[NBUF:]
  wid = lax.axis_index("s") * NC + lax.axis_index("c")
  samp0 = wid * SAMP_PER_W

  pltpu.sync_copy(tok_hbm.at[wid], idx_v)

  def start_gather(c, b):
    pltpu.async_copy(w_hbm.at[idx_v.at[c]], rows_v.at[b], gsems[b])

  def wb_descs(c, b):
    s = samp0 + c * S
    return [
        pltpu.make_async_copy(
            rows_v.at[b, pl.ds(k * SEQ, SEQ)], out_hbm.at[s + k], wsems[b])
        for k in range(S)
    ]

  for b in range(NBUF):
    start_gather(b, b)

  @pl.loop(0, NCH, step=NBUF)
  def _(g0):
    for b in range(NBUF):
      g = g0 + b
      pltpu.make_async_copy(w_hbm.at[idx_v.at[g]], rows_v.at[b], gsems[b]).wait()
      for d in wb_descs(g, b):
        d.start()
      pb = (b - 1) % NBUF

      @pl.when(g > 0)
      def _():
        for d in wb_descs(g - 1, pb):
          d.wait()
      nxt = g - 1 + NBUF

      @pl.when((g > 0) & (nxt < NCH))
      def _():
        start_gather(nxt, pb)

  for d in wb_descs(NCH - 1, (NCH - 1) % NBUF):
    d.wait()


@jax.jit
def kernel(token_ids, weight):
  tok = token_ids.reshape(NW, NCH, CH_IDX).astype(jnp.int32)
  mesh = plsc.VectorSubcoreMesh(core_axis_name="c", subcore_axis_name="s")
  call = pl.kernel(
      _body,
      out_type=jax.ShapeDtypeStruct((NSAMP, SEQ, D), jnp.float32),
      mesh=mesh,
      scratch_types=[
          pltpu.VMEM((NCH, CH_IDX), jnp.int32),
          pltpu.VMEM((NBUF, CH_IDX, D), jnp.float32),
      ] + [pltpu.SemaphoreType.DMA] * (2 * NBUF),
      compiler_params=pltpu.CompilerParams(use_tc_tiling_on_sc=True),
  )
  return call(tok, weight)


# speedup vs baseline: 3.1232x; 1.7499x over previous
import jax
import jax.numpy as jnp
from jax import lax
from jax.experimental import pallas as pl
from jax.experimental.pallas import tpu as pltpu
from jax.experimental.pallas import tpu_sc as plsc

_info = plsc.get_sparse_core_info()
NC, NS = _info.num_cores, _info.num_subcores
NW = NC * NS

NSAMP = 4096
SEQ = 50
D = 128
B = NSAMP * SEQ
ROWS_PER_W = B // NW
CH = 128
NCH = ROWS_PER_W // CH
NBUF = 5


def _body(tok_hbm, w_hbm, out_hbm, idx_v, rows_v, *sems):
  gsems, wsems = sems[:NBUF], sems[NBUF:]
  wid = lax.axis_index("s") * NC + lax.axis_index("c")
  base = wid * ROWS_PER_W

  pltpu.sync_copy(tok_hbm.at[wid], idx_v)

  def start_gather(c, b):
    pltpu.async_copy(
        w_hbm.at[idx_v.at[pl.ds(c * CH, CH)]], rows_v.at[b], gsems[b])

  def wb_desc(c, b):
    return pltpu.make_async_copy(
        rows_v.at[b], out_hbm.at[pl.ds(base + c * CH, CH)], wsems[b])

  for b in range(NBUF):
    start_gather(b, b)

  @pl.loop(0, NCH, step=NBUF)
  def _(g0):
    for b in range(NBUF):
      g = g0 + b
      pltpu.make_async_copy(
          w_hbm.at[idx_v.at[pl.ds(g * CH, CH)]], rows_v.at[b], gsems[b]).wait()
      wb_desc(g, b).start()
      pb = (b - 1) % NBUF

      @pl.when(g > 0)
      def _():
        wb_desc(g - 1, pb).wait()
      nxt = g - 1 + NBUF

      @pl.when((g > 0) & (nxt < NCH))
      def _():
        start_gather(nxt, pb)

  wb_desc(NCH - 1, (NCH - 1) % NBUF).wait()


@jax.jit
def kernel(token_ids, weight):
  tok = token_ids.T.reshape(NW, ROWS_PER_W).astype(jnp.int32)
  mesh = plsc.VectorSubcoreMesh(core_axis_name="c", subcore_axis_name="s")
  call = pl.kernel(
      _body,
      out_type=jax.ShapeDtypeStruct((B, D), jnp.float32),
      mesh=mesh,
      scratch_types=[
          pltpu.VMEM((ROWS_PER_W,), jnp.int32),
          pltpu.VMEM((NBUF, CH, D), jnp.float32),
      ] + [pltpu.SemaphoreType.DMA] * (2 * NBUF),
      compiler_params=pltpu.CompilerParams(use_tc_tiling_on_sc=True),
  )
  out = call(tok, weight)
  return out.reshape(SEQ, NSAMP, D).transpose(1, 0, 2)


# speedup vs baseline: 3.1246x; 1.0004x over previous
import jax
import jax.numpy as jnp
from jax import lax
from jax.experimental import pallas as pl
from jax.experimental.pallas import tpu as pltpu
from jax.experimental.pallas import tpu_sc as plsc

_info = plsc.get_sparse_core_info()
NC, NS = _info.num_cores, _info.num_subcores
NW = NC * NS

NSAMP = 4096
SEQ = 50
D = 128
B = NSAMP * SEQ
ROWS_PER_W = B // NW
CH = 128
NCH = ROWS_PER_W // CH
NBUF = 5


def _body(tok_hbm, w_hbm, out_hbm, idx_v, rows_v, *sems):
  gsems, wsems = sems[:NBUF], sems[NBUF:]
  wid = lax.axis_index("s") * NC + lax.axis_index("c")
  base = wid * ROWS_PER_W

  pltpu.sync_copy(tok_hbm.at[wid], idx_v)

  def start_gather(c, b):
    pltpu.async_copy(w_hbm.at[idx_v.at[c]], rows_v.at[b], gsems[b])

  def wb_desc(c, b):
    return pltpu.make_async_copy(
        rows_v.at[b], out_hbm.at[pl.ds(base + c * CH, CH)], wsems[b])

  for b in range(NBUF):
    start_gather(b, b)

  @pl.loop(0, NCH, step=NBUF)
  def _(g0):
    for b in range(NBUF):
      g = g0 + b
      pltpu.make_async_copy(w_hbm.at[idx_v.at[g]], rows_v.at[b], gsems[b]).wait()
      wb_desc(g, b).start()
      pb = (b - 1) % NBUF

      @pl.when(g > 0)
      def _():
        wb_desc(g - 1, pb).wait()
      nxt = g - 1 + NBUF

      @pl.when((g > 0) & (nxt < NCH))
      def _():
        start_gather(nxt, pb)

  wb_desc(NCH - 1, (NCH - 1) % NBUF).wait()


@jax.jit
def kernel(token_ids, weight):
  tok = token_ids.T.reshape(NW, NCH, CH).astype(jnp.int32)
  mesh = plsc.VectorSubcoreMesh(core_axis_name="c", subcore_axis_name="s")
  call = pl.kernel(
      _body,
      out_type=jax.ShapeDtypeStruct((B, D), jnp.float32),
      mesh=mesh,
      scratch_types=[
          pltpu.VMEM((NCH, CH), jnp.int32),
          pltpu.VMEM((NBUF, CH, D), jnp.float32),
      ] + [pltpu.SemaphoreType.DMA] * (2 * NBUF),
      compiler_params=pltpu.CompilerParams(use_tc_tiling_on_sc=True),
  )
  out = call(tok, weight)
  return out.reshape(SEQ, NSAMP, D).transpose(1, 0, 2)


# speedup vs baseline: 3.1389x; 1.0046x over previous
import jax
import jax.numpy as jnp
from jax import lax
from jax.experimental import pallas as pl
from jax.experimental.pallas import tpu as pltpu
from jax.experimental.pallas import tpu_sc as plsc

_info = plsc.get_sparse_core_info()
NC, NS = _info.num_cores, _info.num_subcores
NW = NC * NS

NSAMP = 4096
SEQ = 50
D = 128
B = NSAMP * SEQ
ROWS_PER_W = B // NW
CH = 64
NCH = ROWS_PER_W // CH
NBUF = 10


def _body(tok_hbm, w_hbm, out_hbm, idx_v, rows_v, *sems):
  gsems, wsems = sems[:NBUF], sems[NBUF:]
  wid = lax.axis_index("s") * NC + lax.axis_index("c")
  base = wid * ROWS_PER_W

  pltpu.sync_copy(tok_hbm.at[wid], idx_v)

  def start_gather(c, b):
    pltpu.async_copy(w_hbm.at[idx_v.at[c]], rows_v.at[b], gsems[b])

  def wb_desc(c, b):
    return pltpu.make_async_copy(
        rows_v.at[b], out_hbm.at[pl.ds(base + c * CH, CH)], wsems[b])

  for b in range(NBUF):
    start_gather(b, b)

  @pl.loop(0, NCH, step=NBUF)
  def _(g0):
    for b in range(NBUF):
      g = g0 + b
      pltpu.make_async_copy(w_hbm.at[idx_v.at[g]], rows_v.at[b], gsems[b]).wait()
      wb_desc(g, b).start()
      pb = (b - 1) % NBUF

      @pl.when(g > 0)
      def _():
        wb_desc(g - 1, pb).wait()
      nxt = g - 1 + NBUF

      @pl.when((g > 0) & (nxt < NCH))
      def _():
        start_gather(nxt, pb)

  wb_desc(NCH - 1, (NCH - 1) % NBUF).wait()


@jax.jit
def kernel(token_ids, weight):
  tok = token_ids.T.reshape(NW, NCH, CH).astype(jnp.int32)
  mesh = plsc.VectorSubcoreMesh(core_axis_name="c", subcore_axis_name="s")
  call = pl.kernel(
      _body,
      out_type=jax.ShapeDtypeStruct((B, D), jnp.float32),
      mesh=mesh,
      scratch_types=[
          pltpu.VMEM((NCH, CH), jnp.int32),
          pltpu.VMEM((NBUF, CH, D), jnp.float32),
      ] + [pltpu.SemaphoreType.DMA] * (2 * NBUF),
      compiler_params=pltpu.CompilerParams(use_tc_tiling_on_sc=True),
  )
  out = call(tok, weight)
  return out.reshape(SEQ, NSAMP, D).transpose(1, 0, 2)


# speedup vs baseline: 3.2200x; 1.0259x over previous
import jax
import jax.numpy as jnp
from jax import lax
from jax.experimental import pallas as pl
from jax.experimental.pallas import tpu as pltpu
from jax.experimental.pallas import tpu_sc as plsc

_info = plsc.get_sparse_core_info()
NC, NS = _info.num_cores, _info.num_subcores
NW = NC * NS

NSAMP = 4096
SEQ = 50
D = 128
B = NSAMP * SEQ
CH = NSAMP // NW
NCH = SEQ
NBUF = 5


def _body(tok_hbm, w_hbm, out_hbm, idx_v, rows_v, *sems):
  gsems, wsems = sems[:NBUF], sems[NBUF:]
  wid = lax.axis_index("s") * NC + lax.axis_index("c")
  col0 = wid * CH

  pltpu.sync_copy(tok_hbm.at[:, pl.ds(col0, CH)], idx_v)

  def start_gather(c, b):
    pltpu.async_copy(w_hbm.at[idx_v.at[c]], rows_v.at[b], gsems[b])

  def wb_desc(c, b):
    return pltpu.make_async_copy(
        rows_v.at[b], out_hbm.at[pl.ds(c * NSAMP + col0, CH)], wsems[b])

  for b in range(NBUF):
    start_gather(b, b)

  @pl.loop(0, NCH, step=NBUF)
  def _(g0):
    for b in range(NBUF):
      g = g0 + b
      pltpu.make_async_copy(w_hbm.at[idx_v.at[g]], rows_v.at[b], gsems[b]).wait()
      wb_desc(g, b).start()
      pb = (b - 1) % NBUF

      @pl.when(g > 0)
      def _():
        wb_desc(g - 1, pb).wait()
      nxt = g - 1 + NBUF

      @pl.when((g > 0) & (nxt < NCH))
      def _():
        start_gather(nxt, pb)

  wb_desc(NCH - 1, (NCH - 1) % NBUF).wait()


@jax.jit
def kernel(token_ids, weight):
  tok = token_ids.T.astype(jnp.int32)
  mesh = plsc.VectorSubcoreMesh(core_axis_name="c", subcore_axis_name="s")
  call = pl.kernel(
      _body,
      out_type=jax.ShapeDtypeStruct((B, D), jnp.float32),
      mesh=mesh,
      scratch_types=[
          pltpu.VMEM((NCH, CH), jnp.int32),
          pltpu.VMEM((NBUF, CH, D), jnp.float32),
      ] + [pltpu.SemaphoreType.DMA] * (2 * NBUF),
      compiler_params=pltpu.CompilerParams(use_tc_tiling_on_sc=True),
  )
  out = call(tok, weight)
  return out.reshape(SEQ, NSAMP, D).transpose(1, 0, 2)
